# baseline (speedup 1.0000x reference)
import functools

import jax
import jax.numpy as jnp
import numpy as np
from jax import lax
from jax.experimental import pallas as pl
from jax.experimental.pallas import tpu as pltpu
from jax.experimental.pallas import tpu_sc as plsc


def _ap(p, x):
    y = x @ p['W']
    if p['b'] is not None:
        y = y + p['b']
    return y


def _cdist(a, b):
    d2 = jnp.sum(a * a, -1)[:, :, None] + jnp.sum(b * b, -1)[:, None, :] - 2.0 * jnp.einsum('bnd,bmd->bnm', a, b)
    return jnp.sqrt(jnp.maximum(d2, 0.0))


def _roll3(x, sh, axis):
    if axis == 1:
        return jnp.concatenate([x[:, sh:, :], x[:, :sh, :]], axis=1)
    return pltpu.roll(x, x.shape[2] - sh, 2)


def _partner3(x, j, ij0):
    L = x.shape[2]
    if j < L:
        lo = _roll3(x, j, 2)
        hi = _roll3(x, L - j, 2)
    else:
        m = j // L
        lo = _roll3(x, m, 1)
        hi = _roll3(x, x.shape[1] - m, 1)
    return jnp.where(ij0, lo, hi)


def _sort_kernel3(n_mid, n_lane, k_ref, v_out):
    k = k_ref[...]
    R = k.shape[0]
    shape = (R, n_mid, n_lane)
    lane = jax.lax.broadcasted_iota(jnp.int32, shape, 2)
    mid = jax.lax.broadcasted_iota(jnp.int32, shape, 1)
    iota = mid * n_lane + lane
    v = iota
    n = n_mid * n_lane
    ksz = 2
    while ksz <= n:
        j = ksz // 2
        while j >= 1:
            ij0 = (iota & j) == 0
            up = (iota & ksz) == 0
            tm = ij0 == up
            kp = _partner3(k, j, ij0)
            vp = _partner3(v, j, ij0)
            le = (k < kp) | ((k == kp) & (v < vp))
            keep = le == tm
            k = jnp.where(keep, k, kp)
            v = jnp.where(keep, v, vp)
            j //= 2
        ksz *= 2
    v_out[...] = v


def _argsort_rows(keys, block_rows=128, n_lane=128):
    M, n = keys.shape
    n_mid = n // n_lane
    k3 = keys.reshape(M, n_mid, n_lane)
    out = pl.pallas_call(
        functools.partial(_sort_kernel3, n_mid, n_lane),
        grid=(M // block_rows,),
        in_specs=[pl.BlockSpec((block_rows, n_mid, n_lane), lambda i: (i, 0, 0))],
        out_specs=pl.BlockSpec((block_rows, n_mid, n_lane), lambda i: (i, 0, 0)),
        out_shape=jax.ShapeDtypeStruct((M, n_mid, n_lane), jnp.int32),
    )(k3)
    return out.reshape(M, n)


def _gather(x, idx):
    return jax.vmap(lambda xb, ib: xb[ib])(x, idx)


def _edge_conv(p, x, idx):
    xj = _gather(x, idx)
    xi = x[:, :, None, :]
    e = jnp.concatenate([jnp.broadcast_to(xi, xj.shape), xj - xi], -1)
    h = jax.nn.relu(_ap(p['m1'], e))
    h = jax.nn.relu(_ap(p['m2'], h))
    return jnp.max(h, axis=2)


def _sc_gather(table, idx):
    V, D = table.shape
    E = idx.shape[0]
    info = plsc.get_sparse_core_info()
    nw = info.num_cores * info.num_subcores
    b_per_w = E // nw
    ch = min(512, b_per_w)
    n_ch = b_per_w // ch
    mesh = plsc.VectorSubcoreMesh(core_axis_name="c", subcore_axis_name="s")

    @functools.partial(
        pl.kernel, mesh=mesh,
        out_type=jax.ShapeDtypeStruct((E, D), jnp.float32),
        scratch_types=[
            pltpu.VMEM((ch,), jnp.int32),
            pltpu.VMEM((ch, D), jnp.float32),
            pltpu.SemaphoreType.DMA,
        ],
    )
    def k(table_hbm, idx_hbm, out_hbm, idx_v, rows_v, sem):
        wid = lax.axis_index("s") * info.num_cores + lax.axis_index("c")
        base = wid * b_per_w
        for c in range(n_ch):
            off = base + c * ch
            pltpu.sync_copy(idx_hbm.at[pl.ds(off, ch)], idx_v)
            pltpu.async_copy(table_hbm.at[idx_v], rows_v, sem).wait()
            pltpu.sync_copy(rows_v, out_hbm.at[pl.ds(off, ch)])

    return k(table, idx)


def _ab_kernel(x_ref, wa_ref, wb_ref, b_ref, a_ref, bm_ref):
    xx = x_ref[...]
    a_ref[...] = jnp.dot(xx, wa_ref[...], preferred_element_type=jnp.float32) + b_ref[...]
    bm_ref[...] = jnp.dot(xx, wb_ref[...], preferred_element_type=jnp.float32)


def _edge_mlp_kernel(a_ref, g_ref, w2_ref, b2_ref, out_ref):
    BM, H = a_ref.shape
    O = w2_ref.shape[1]
    g = g_ref[...]
    e3 = g.reshape(BM, 32, H) + a_ref[...][:, None, :]
    e = jax.nn.relu(e3).reshape(BM * 32, H)
    h = jnp.dot(e, w2_ref[...], preferred_element_type=jnp.float32) + b2_ref[...]
    out_ref[...] = jnp.max(jax.nn.relu(h).reshape(BM, 32, O), axis=1)


def _pad_to(w, r, c):
    return jnp.zeros((r, c), jnp.float32).at[:w.shape[0], :w.shape[1]].set(w)


def _edge_conv_fused(p, x, idx_glob):
    B, N, C = x.shape
    H = p['m1']['W'].shape[1]
    O = p['m2']['W'].shape[1]
    M = B * N
    W1 = p['m1']['W']
    W1a, W1b = W1[:C], W1[C:]
    wa = _pad_to(W1a - W1b, C, 128)
    wb = _pad_to(W1b, C, 128)
    b1p = jnp.zeros((1, 128), jnp.float32).at[0, :H].set(p['m1']['b'])
    a, bm = pl.pallas_call(
        _ab_kernel,
        in_specs=[pl.BlockSpec((M, C), lambda: (0, 0)),
                  pl.BlockSpec((C, 128), lambda: (0, 0)),
                  pl.BlockSpec((C, 128), lambda: (0, 0)),
                  pl.BlockSpec((1, 128), lambda: (0, 0))],
        out_specs=[pl.BlockSpec((M, 128), lambda: (0, 0)),
                   pl.BlockSpec((M, 128), lambda: (0, 0))],
        out_shape=[jax.ShapeDtypeStruct((M, 128), jnp.float32),
                   jax.ShapeDtypeStruct((M, 128), jnp.float32)],
    )(x.reshape(M, C), wa, wb, b1p)
    g = _sc_gather(bm, idx_glob)
    w2p = _pad_to(p['m2']['W'], 128, 128)
    b2p = jnp.zeros((1, 128), jnp.float32).at[0, :O].set(p['m2']['b'])
    BM = 256
    out = pl.pallas_call(
        _edge_mlp_kernel,
        grid=(M // BM,),
        in_specs=[pl.BlockSpec((BM, 128), lambda i: (i, 0)),
                  pl.BlockSpec((BM * 32, 128), lambda i: (i, 0)),
                  pl.BlockSpec((128, 128), lambda i: (0, 0)),
                  pl.BlockSpec((1, 128), lambda i: (0, 0))],
        out_specs=pl.BlockSpec((BM, 128), lambda i: (i, 0)),
        out_shape=jax.ShapeDtypeStruct((M, 128), jnp.float32),
    )(a, g, w2p, b2p)
    return out[:, :O].reshape(B, N, O)


def _flat_idx(idx):
    B, N, K = idx.shape
    off = (jnp.arange(B, dtype=jnp.int32) * N)[:, None, None]
    return (idx.astype(jnp.int32) + off).reshape(B * N * K)


def _ln(p, x):
    m = jnp.mean(x, -1, keepdims=True)
    v = jnp.var(x, -1, keepdims=True)
    return (x - m) / jnp.sqrt(v + 1e-5) * p['g'] + p['b']


def _stn(p, x):
    h = jax.nn.relu(_ap(p['c1'], x))
    h = jax.nn.relu(_ap(p['c2'], h))
    h = jax.nn.relu(_ap(p['c3'], h))
    g = jnp.max(h, axis=1)
    g = jax.nn.relu(_ap(p['f1'], g))
    g = jax.nn.relu(_ap(p['f2'], g))
    t = _ap(p['f3'], g) + jnp.eye(24, dtype=jnp.float32).reshape(-1)
    t = t.reshape(-1, 24, 24)
    return jnp.einsum('bnc,bcd->bnd', x, t)


def _top32_kernel(n_mid, n_lane, k_ref, out_ref):
    k = k_ref[...]
    R = k.shape[0]
    shape = (R, n_mid, n_lane)
    lane = jax.lax.broadcasted_iota(jnp.int32, shape, 2)
    mid = jax.lax.broadcasted_iota(jnp.int32, shape, 1)
    v = mid * n_lane + lane
    big = jnp.int32(1 << 30)
    lane2 = jax.lax.broadcasted_iota(jnp.int32, (R, 128), 1)
    outv = jnp.zeros((R, 128), jnp.int32)
    for t in range(32):
        m = jnp.min(jnp.min(k, axis=2, keepdims=True), axis=1, keepdims=True)
        cand = jnp.where(k == m, v, big)
        vm = jnp.min(jnp.min(cand, axis=2, keepdims=True), axis=1, keepdims=True)
        outv = jnp.where(lane2 == t, jnp.broadcast_to(vm[:, :, 0], (R, 128)), outv)
        k = jnp.where(v == vm, jnp.inf, k)
    out_ref[...] = outv


def _top32_idx(cdm, block_rows=128, n_lane=128):
    B, N, _ = cdm.shape
    M = B * N
    n_mid = N // n_lane
    k3 = cdm.reshape(M, n_mid, n_lane)
    out = pl.pallas_call(
        functools.partial(_top32_kernel, n_mid, n_lane),
        grid=(M // block_rows,),
        in_specs=[pl.BlockSpec((block_rows, n_mid, n_lane), lambda i: (i, 0, 0))],
        out_specs=pl.BlockSpec((block_rows, 128), lambda i: (i, 0)),
        out_shape=jax.ShapeDtypeStruct((M, 128), jnp.int32),
    )(k3)
    return out[:, :32].reshape(B, N, 32)


def _head_kernel(xf_ref, wfi_ref,
                 w1a_ref, b1a_ref, w1b_ref, b1b_ref, w1r_ref, b1r_ref,
                 w2a_ref, b2a_ref, w2b_ref, b2b_ref, w2r_ref, b2r_ref,
                 wo_ref, bo_ref,
                 seg_ref, feat_ref):
    xf = xf_ref[...]
    xg = xf * jax.nn.sigmoid(jnp.dot(xf, wfi_ref[...], preferred_element_type=jnp.float32))
    h = jax.nn.relu(jnp.dot(xg, w1a_ref[...], preferred_element_type=jnp.float32) + b1a_ref[...])
    r1 = jax.nn.relu(jnp.dot(h, w1b_ref[...], preferred_element_type=jnp.float32) + b1b_ref[...])
    r1 = r1 + jnp.dot(xg, w1r_ref[...], preferred_element_type=jnp.float32) + b1r_ref[...]
    h2 = jax.nn.relu(jnp.dot(r1, w2a_ref[...], preferred_element_type=jnp.float32) + b2a_ref[...])
    feat = jax.nn.relu(jnp.dot(h2, w2b_ref[...], preferred_element_type=jnp.float32) + b2b_ref[...])
    feat = feat + jnp.dot(r1, w2r_ref[...], preferred_element_type=jnp.float32) + b2r_ref[...]
    feat_ref[...] = feat
    seg_ref[...] = jnp.dot(feat, wo_ref[...], preferred_element_type=jnp.float32) + bo_ref[...]


def _run_head(x_fused, params):
    B, N = x_fused.shape[0], x_fused.shape[1]
    M = B * N
    xf = x_fused.reshape(M, 256)
    p = params
    wo = jnp.zeros((256, 128), jnp.float32).at[:, :17].set(p['out']['W'])
    bo = jnp.zeros((1, 128), jnp.float32).at[0, :17].set(p['out']['b'])
    BM = 1024
    grid = (M // BM,)
    row_spec = lambda c: pl.BlockSpec((BM, c), lambda i: (i, 0))
    args = [xf,
            p['fi']['W'],
            p['rb1a']['W'], p['rb1a']['b'].reshape(1, -1),
            p['rb1b']['W'], p['rb1b']['b'].reshape(1, -1),
            p['rb1r']['W'], p['rb1r']['b'].reshape(1, -1),
            p['rb2a']['W'], p['rb2a']['b'].reshape(1, -1),
            p['rb2b']['W'], p['rb2b']['b'].reshape(1, -1),
            p['rb2r']['W'], p['rb2r']['b'].reshape(1, -1),
            wo, bo]
    full = lambda a: pl.BlockSpec(a.shape, lambda i: tuple(0 for _ in a.shape))
    in_specs = [row_spec(256)] + [full(a) for a in args[1:]]
    seg, feat = pl.pallas_call(
        _head_kernel,
        grid=grid,
        in_specs=in_specs,
        out_specs=[row_spec(128), row_spec(256)],
        out_shape=[jax.ShapeDtypeStruct((M, 128), jnp.float32),
                   jax.ShapeDtypeStruct((M, 256), jnp.float32)],
    )(*args)
    return seg[:, :17].reshape(B, N, 17), feat.reshape(B, N, 256)


def kernel(x, pos, labels, params):
    B, N = x.shape[0], x.shape[1]
    cd = _cdist(pos, pos)
    sidx = _argsort_rows(cd.reshape(B * N, N))
    knn32 = sidx[:, :32].reshape(B, N, 32)
    dil200 = sidx[:, ::6][:, :32].reshape(B, N, 32)
    dil900 = sidx[:, ::28][:, :32].reshape(B, N, 32)
    dil1800 = sidx[:, ::56][:, :32].reshape(B, N, 32)
    nidx = sidx[:, 1:9].reshape(B, N, 8)
    x = _stn(params['stn'], x)
    x1 = _edge_conv_fused(params['e1'], x, _flat_idx(knn32))
    x2 = _edge_conv_fused(params['e2'], x1, _flat_idx(_top32_idx(_cdist(x1, x1))))
    x3 = _edge_conv_fused(params['e3'], x2, _flat_idx(_top32_idx(_cdist(x2, x2))))
    x_local = jnp.concatenate([x1, x2, x3], -1)
    x_mid = jax.nn.relu(_ap(params['local_hidden'], x_local))
    xd1 = _edge_conv_fused(params['d1'], x_mid, _flat_idx(dil200))
    xd2 = _edge_conv_fused(params['d2'], xd1, _flat_idx(dil900))
    xd3 = _edge_conv_fused(params['d3'], xd2, _flat_idx(dil1800))
    x_global = jnp.concatenate([xd1, xd2, xd3], -1)
    x_temp = jnp.concatenate([x_mid, xd1, xd2, xd3], -1)
    logits_temp = _ap(params['temp2'], jax.nn.relu(_ln(params['temp_ln'], _ap(params['temp1'], x_temp))))
    f0 = _ap(params['proj0'], x_local)
    f1 = _ap(params['proj1'], x_mid)
    f2 = _ap(params['proj2'], x_global)
    fs = jnp.stack([f0, f1, f2], axis=2)
    tl = jnp.argmax(logits_temp, -1)
    nl = jax.vmap(lambda lb, ib: lb[ib])(tl, nidx)
    diff = jnp.mean((nl != tl[:, :, None]).astype(jnp.float32), -1)
    probs = jax.nn.softmax(logits_temp, -1)
    conf = jnp.max(probs, -1)
    ent = -jnp.sum(probs * jnp.log(probs + 1e-8), -1) / np.log(probs.shape[-1])
    binfo = jnp.stack([diff, conf, ent], -1)
    benc = _ap(params['be2'], jax.nn.relu(_ap(params['be1'], binfo)))
    gfeat = jnp.mean(fs, axis=2)
    aw = jax.nn.softmax(_ap(params['at2'], jax.nn.relu(_ap(params['at1'], jnp.concatenate([gfeat, benc], -1)))), -1)
    fused = jnp.sum(fs * aw[:, :, :, None], axis=2)
    x_fused = _ap(params['op2'], jax.nn.relu(_ap(params['op1'], fused))) + gfeat
    seg_pred, features = _run_head(x_fused, params)
    return (seg_pred, features, x_fused)


# speedup vs baseline: 1.0274x; 1.0274x over previous
import functools

import jax
import jax.numpy as jnp
import numpy as np
from jax import lax
from jax.experimental import pallas as pl
from jax.experimental.pallas import tpu as pltpu
from jax.experimental.pallas import tpu_sc as plsc


def _ap(p, x):
    y = x @ p['W']
    if p['b'] is not None:
        y = y + p['b']
    return y


def _cdist(a, b):
    d2 = jnp.sum(a * a, -1)[:, :, None] + jnp.sum(b * b, -1)[:, None, :] - 2.0 * jnp.einsum('bnd,bmd->bnm', a, b)
    return jnp.sqrt(jnp.maximum(d2, 0.0))


def _roll3(x, sh, axis):
    if axis == 1:
        return jnp.concatenate([x[:, sh:, :], x[:, :sh, :]], axis=1)
    return pltpu.roll(x, x.shape[2] - sh, 2)


def _partner3(x, j, ij0):
    L = x.shape[2]
    if j < L:
        lo = _roll3(x, j, 2)
        hi = _roll3(x, L - j, 2)
    else:
        m = j // L
        lo = _roll3(x, m, 1)
        hi = _roll3(x, x.shape[1] - m, 1)
    return jnp.where(ij0, lo, hi)


def _sort_kernel3(n_mid, n_lane, k_ref, v_out):
    k = k_ref[...]
    R = k.shape[0]
    shape = (R, n_mid, n_lane)
    lane = jax.lax.broadcasted_iota(jnp.int32, shape, 2)
    mid = jax.lax.broadcasted_iota(jnp.int32, shape, 1)
    iota = mid * n_lane + lane
    v = iota
    n = n_mid * n_lane
    ksz = 2
    while ksz <= n:
        j = ksz // 2
        while j >= 1:
            ij0 = (iota & j) == 0
            up = (iota & ksz) == 0
            tm = ij0 == up
            kp = _partner3(k, j, ij0)
            vp = _partner3(v, j, ij0)
            le = (k < kp) | ((k == kp) & (v < vp))
            keep = le == tm
            k = jnp.where(keep, k, kp)
            v = jnp.where(keep, v, vp)
            j //= 2
        ksz *= 2
    v_out[...] = v


def _argsort_rows(keys, block_rows=256, n_lane=128):
    M, n = keys.shape
    n_mid = n // n_lane
    k3 = keys.reshape(M, n_mid, n_lane)
    out = pl.pallas_call(
        functools.partial(_sort_kernel3, n_mid, n_lane),
        grid=(M // block_rows,),
        in_specs=[pl.BlockSpec((block_rows, n_mid, n_lane), lambda i: (i, 0, 0))],
        out_specs=pl.BlockSpec((block_rows, n_mid, n_lane), lambda i: (i, 0, 0)),
        out_shape=jax.ShapeDtypeStruct((M, n_mid, n_lane), jnp.int32),
    )(k3)
    return out.reshape(M, n)


def _gather(x, idx):
    return jax.vmap(lambda xb, ib: xb[ib])(x, idx)


def _edge_conv(p, x, idx):
    xj = _gather(x, idx)
    xi = x[:, :, None, :]
    e = jnp.concatenate([jnp.broadcast_to(xi, xj.shape), xj - xi], -1)
    h = jax.nn.relu(_ap(p['m1'], e))
    h = jax.nn.relu(_ap(p['m2'], h))
    return jnp.max(h, axis=2)


def _sc_gather(table, idx):
    V, D = table.shape
    E = idx.shape[0]
    info = plsc.get_sparse_core_info()
    nw = info.num_cores * info.num_subcores
    b_per_w = E // nw
    ch = min(512, b_per_w)
    n_ch = b_per_w // ch
    mesh = plsc.VectorSubcoreMesh(core_axis_name="c", subcore_axis_name="s")

    @functools.partial(
        pl.kernel, mesh=mesh,
        out_type=jax.ShapeDtypeStruct((E, D), jnp.float32),
        scratch_types=[
            pltpu.VMEM((ch,), jnp.int32),
            pltpu.VMEM((ch, D), jnp.float32),
            pltpu.SemaphoreType.DMA,
        ],
    )
    def k(table_hbm, idx_hbm, out_hbm, idx_v, rows_v, sem):
        wid = lax.axis_index("s") * info.num_cores + lax.axis_index("c")
        base = wid * b_per_w
        for c in range(n_ch):
            off = base + c * ch
            pltpu.sync_copy(idx_hbm.at[pl.ds(off, ch)], idx_v)
            pltpu.async_copy(table_hbm.at[idx_v], rows_v, sem).wait()
            pltpu.sync_copy(rows_v, out_hbm.at[pl.ds(off, ch)])

    return k(table, idx)


def _ab_kernel(x_ref, wa_ref, wb_ref, b_ref, a_ref, bm_ref):
    xx = x_ref[...]
    a_ref[...] = jnp.dot(xx, wa_ref[...], preferred_element_type=jnp.float32) + b_ref[...]
    bm_ref[...] = jnp.dot(xx, wb_ref[...], preferred_element_type=jnp.float32)


def _edge_mlp_kernel(a_ref, g_ref, w2_ref, b2_ref, out_ref):
    BM, H = a_ref.shape
    O = w2_ref.shape[1]
    g = g_ref[...]
    e3 = g.reshape(BM, 32, H) + a_ref[...][:, None, :]
    e = jax.nn.relu(e3).reshape(BM * 32, H)
    h = jnp.dot(e, w2_ref[...], preferred_element_type=jnp.float32) + b2_ref[...]
    out_ref[...] = jnp.max(jax.nn.relu(h).reshape(BM, 32, O), axis=1)


def _pad_to(w, r, c):
    return jnp.zeros((r, c), jnp.float32).at[:w.shape[0], :w.shape[1]].set(w)


def _edge_conv_fused(p, x, idx_glob):
    B, N, C = x.shape
    H = p['m1']['W'].shape[1]
    O = p['m2']['W'].shape[1]
    M = B * N
    W1 = p['m1']['W']
    W1a, W1b = W1[:C], W1[C:]
    wa = _pad_to(W1a - W1b, C, 128)
    wb = _pad_to(W1b, C, 128)
    b1p = jnp.zeros((1, 128), jnp.float32).at[0, :H].set(p['m1']['b'])
    a, bm = pl.pallas_call(
        _ab_kernel,
        in_specs=[pl.BlockSpec((M, C), lambda: (0, 0)),
                  pl.BlockSpec((C, 128), lambda: (0, 0)),
                  pl.BlockSpec((C, 128), lambda: (0, 0)),
                  pl.BlockSpec((1, 128), lambda: (0, 0))],
        out_specs=[pl.BlockSpec((M, 128), lambda: (0, 0)),
                   pl.BlockSpec((M, 128), lambda: (0, 0))],
        out_shape=[jax.ShapeDtypeStruct((M, 128), jnp.float32),
                   jax.ShapeDtypeStruct((M, 128), jnp.float32)],
    )(x.reshape(M, C), wa, wb, b1p)
    g = _sc_gather(bm, idx_glob)
    w2p = _pad_to(p['m2']['W'], 128, 128)
    b2p = jnp.zeros((1, 128), jnp.float32).at[0, :O].set(p['m2']['b'])
    BM = 256
    out = pl.pallas_call(
        _edge_mlp_kernel,
        grid=(M // BM,),
        in_specs=[pl.BlockSpec((BM, 128), lambda i: (i, 0)),
                  pl.BlockSpec((BM * 32, 128), lambda i: (i, 0)),
                  pl.BlockSpec((128, 128), lambda i: (0, 0)),
                  pl.BlockSpec((1, 128), lambda i: (0, 0))],
        out_specs=pl.BlockSpec((BM, 128), lambda i: (i, 0)),
        out_shape=jax.ShapeDtypeStruct((M, 128), jnp.float32),
    )(a, g, w2p, b2p)
    return out


def _flat_idx(idx):
    B, N, K = idx.shape
    off = (jnp.arange(B, dtype=jnp.int32) * N)[:, None, None]
    return (idx.astype(jnp.int32) + off).reshape(B * N * K)


def _ln(p, x):
    m = jnp.mean(x, -1, keepdims=True)
    v = jnp.var(x, -1, keepdims=True)
    return (x - m) / jnp.sqrt(v + 1e-5) * p['g'] + p['b']


def _stn(p, x):
    h = jax.nn.relu(_ap(p['c1'], x))
    h = jax.nn.relu(_ap(p['c2'], h))
    h = jax.nn.relu(_ap(p['c3'], h))
    g = jnp.max(h, axis=1)
    g = jax.nn.relu(_ap(p['f1'], g))
    g = jax.nn.relu(_ap(p['f2'], g))
    t = _ap(p['f3'], g) + jnp.eye(24, dtype=jnp.float32).reshape(-1)
    t = t.reshape(-1, 24, 24)
    return jnp.einsum('bnc,bcd->bnd', x, t)


def _top32_kernel(n_mid, n_lane, k_ref, out_ref):
    k = k_ref[...]
    R = k.shape[0]
    shape = (R, n_mid, n_lane)
    lane = jax.lax.broadcasted_iota(jnp.int32, shape, 2)
    mid = jax.lax.broadcasted_iota(jnp.int32, shape, 1)
    v = mid * n_lane + lane
    big = jnp.int32(1 << 30)
    lane2 = jax.lax.broadcasted_iota(jnp.int32, (R, 128), 1)
    outv = jnp.zeros((R, 128), jnp.int32)
    for t in range(32):
        m = jnp.min(jnp.min(k, axis=2, keepdims=True), axis=1, keepdims=True)
        cand = jnp.where(k == m, v, big)
        vm = jnp.min(jnp.min(cand, axis=2, keepdims=True), axis=1, keepdims=True)
        outv = jnp.where(lane2 == t, jnp.broadcast_to(vm[:, :, 0], (R, 128)), outv)
        k = jnp.where(v == vm, jnp.inf, k)
    out_ref[...] = outv


def _top32_idx(cdm, block_rows=128, n_lane=128):
    B, N, _ = cdm.shape
    M = B * N
    n_mid = N // n_lane
    k3 = cdm.reshape(M, n_mid, n_lane)
    out = pl.pallas_call(
        functools.partial(_top32_kernel, n_mid, n_lane),
        grid=(M // block_rows,),
        in_specs=[pl.BlockSpec((block_rows, n_mid, n_lane), lambda i: (i, 0, 0))],
        out_specs=pl.BlockSpec((block_rows, 128), lambda i: (i, 0)),
        out_shape=jax.ShapeDtypeStruct((M, 128), jnp.int32),
    )(k3)
    return out[:, :32].reshape(B, N, 32)


def _mid_kernel(xl_ref, w_ref, b_ref, out_ref):
    out_ref[...] = jax.nn.relu(
        jnp.dot(xl_ref[...], w_ref[...], preferred_element_type=jnp.float32) + b_ref[...])


def _h1_kernel(xm_ref, x1_ref, x2_ref, x3_ref,
               w0_ref, w1_ref, w2_ref, w3_ref, bt_ref,
               g_ref, lb_ref, wt2_ref, bt2_ref, out_ref):
    dot = lambda a, w: jnp.dot(a, w, preferred_element_type=jnp.float32)
    t = (dot(xm_ref[...], w0_ref[...]) + dot(x1_ref[...], w1_ref[...])
         + dot(x2_ref[...], w2_ref[...]) + dot(x3_ref[...], w3_ref[...]) + bt_ref[...])
    m = jnp.mean(t, -1, keepdims=True)
    v = jnp.mean((t - m) * (t - m), -1, keepdims=True)
    t = (t - m) / jnp.sqrt(v + 1e-5) * g_ref[...] + lb_ref[...]
    l = dot(jax.nn.relu(t), wt2_ref[...]) + bt2_ref[...]
    BM = l.shape[0]
    lane = jax.lax.broadcasted_iota(jnp.int32, (BM, 128), 1)
    l = jnp.where(lane < 17, l, -1e30)
    mx = jnp.max(l, -1, keepdims=True)
    tl = jnp.min(jnp.where(l == mx, lane, 1 << 30), -1, keepdims=True)
    p = jnp.exp(l - mx)
    probs = p / jnp.sum(p, -1, keepdims=True)
    conf = jnp.max(probs, -1, keepdims=True)
    ent = -jnp.sum(probs * jnp.log(probs + 1e-8), -1, keepdims=True) / np.log(17.0)
    out = jnp.where(lane == 0, tl.astype(jnp.float32),
                    jnp.where(lane == 1, conf, jnp.where(lane == 2, ent, 0.0)))
    out_ref[...] = out


def _h2_kernel(xl_ref, xm_ref, x1_ref, x2_ref, x3_ref, bi_ref,
               wp0_ref, bp0_ref, wp1_ref, bp1_ref,
               wp2a_ref, wp2b_ref, wp2c_ref, bp2_ref,
               wbe1_ref, bbe1_ref, wbe2_ref, bbe2_ref,
               wat1a_ref, wat1b_ref, bat1_ref, wat2_ref, bat2_ref,
               wop1_ref, bop1_ref, wop2_ref, bop2_ref,
               wfi_ref,
               w1a_ref, b1a_ref, w1b_ref, b1b_ref, w1r_ref, b1r_ref,
               w2a_ref, b2a_ref, w2b_ref, b2b_ref, w2r_ref, b2r_ref,
               wo_ref, bo_ref,
               seg_ref, feat_ref, xf_ref):
    dot = lambda a, w: jnp.dot(a, w, preferred_element_type=jnp.float32)
    f0 = dot(xl_ref[...], wp0_ref[...]) + bp0_ref[...]
    f1 = dot(xm_ref[...], wp1_ref[...]) + bp1_ref[...]
    f2 = (dot(x1_ref[...], wp2a_ref[...]) + dot(x2_ref[...], wp2b_ref[...])
          + dot(x3_ref[...], wp2c_ref[...]) + bp2_ref[...])
    gfeat = (f0 + f1 + f2) / 3.0
    benc = dot(jax.nn.relu(dot(bi_ref[...], wbe1_ref[...]) + bbe1_ref[...]), wbe2_ref[...]) + bbe2_ref[...]
    a1 = jax.nn.relu(dot(gfeat, wat1a_ref[...]) + dot(benc, wat1b_ref[...]) + bat1_ref[...])
    al = dot(a1, wat2_ref[...]) + bat2_ref[...]
    BM = al.shape[0]
    lane = jax.lax.broadcasted_iota(jnp.int32, (BM, 128), 1)
    al = jnp.where(lane < 3, al, -1e30)
    p = jnp.exp(al - jnp.max(al, -1, keepdims=True))
    aw = p / jnp.sum(p, -1, keepdims=True)
    pick = lambda k: jnp.sum(jnp.where(lane == k, aw, 0.0), -1, keepdims=True)
    fused = f0 * pick(0) + f1 * pick(1) + f2 * pick(2)
    xf = dot(jax.nn.relu(dot(fused, wop1_ref[...]) + bop1_ref[...]), wop2_ref[...]) + bop2_ref[...] + gfeat
    xf_ref[...] = xf
    xg = xf * jax.nn.sigmoid(dot(xf, wfi_ref[...]))
    h = jax.nn.relu(dot(xg, w1a_ref[...]) + b1a_ref[...])
    r1 = jax.nn.relu(dot(h, w1b_ref[...]) + b1b_ref[...])
    r1 = r1 + dot(xg, w1r_ref[...]) + b1r_ref[...]
    h2 = jax.nn.relu(dot(r1, w2a_ref[...]) + b2a_ref[...])
    feat = jax.nn.relu(dot(h2, w2b_ref[...]) + b2b_ref[...])
    feat = feat + dot(r1, w2r_ref[...]) + b2r_ref[...]
    feat_ref[...] = feat
    seg_ref[...] = dot(feat, wo_ref[...]) + bo_ref[...]


def _bias_pad(b, n=128):
    return jnp.zeros((1, n), jnp.float32).at[0, :b.shape[0]].set(b)


def _run_mid(x_local, p):
    M = x_local.shape[0]
    w = _pad_to(p['local_hidden']['W'], 72, 128)
    b = _bias_pad(p['local_hidden']['b'])
    BM = 512
    return pl.pallas_call(
        _mid_kernel,
        grid=(M // BM,),
        in_specs=[pl.BlockSpec((BM, 72), lambda i: (i, 0)),
                  pl.BlockSpec((72, 128), lambda i: (0, 0)),
                  pl.BlockSpec((1, 128), lambda i: (0, 0))],
        out_specs=pl.BlockSpec((BM, 128), lambda i: (i, 0)),
        out_shape=jax.ShapeDtypeStruct((M, 128), jnp.float32),
    )(x_local, w, b)


def _run_h1(xm_p, xd1_p, xd2_p, xd3_p, p):
    M = xm_p.shape[0]
    wt = p['temp1']['W']
    w0 = _pad_to(wt[:60], 128, 128)
    w1 = _pad_to(wt[60:120], 128, 128)
    w2 = _pad_to(wt[120:180], 128, 128)
    w3 = _pad_to(wt[180:240], 128, 128)
    bt = _bias_pad(p['temp1']['b'])
    g = _bias_pad(p['temp_ln']['g'])
    lb = _bias_pad(p['temp_ln']['b'])
    wt2 = _pad_to(p['temp2']['W'], 128, 128)
    bt2 = _bias_pad(p['temp2']['b'])
    BM = 512
    full = lambda a: pl.BlockSpec(a.shape, lambda i: tuple(0 for _ in a.shape))
    args = [xm_p, xd1_p, xd2_p, xd3_p, w0, w1, w2, w3, bt, g, lb, wt2, bt2]
    in_specs = [pl.BlockSpec((BM, 128), lambda i: (i, 0))] * 4 + [full(a) for a in args[4:]]
    return pl.pallas_call(
        _h1_kernel,
        grid=(M // BM,),
        in_specs=in_specs,
        out_specs=pl.BlockSpec((BM, 128), lambda i: (i, 0)),
        out_shape=jax.ShapeDtypeStruct((M, 128), jnp.float32),
    )(*args)


def _run_h2(x_local, xm_p, xd1_p, xd2_p, xd3_p, binfo, p):
    M = x_local.shape[0]
    wat1 = p['at1']['W']
    args = [
        x_local, xm_p, xd1_p, xd2_p, xd3_p, binfo,
        p['proj0']['W'], _bias_pad(p['proj0']['b'], 256),
        _pad_to(p['proj1']['W'], 128, 256), _bias_pad(p['proj1']['b'], 256),
        _pad_to(p['proj2']['W'][:60], 128, 256), _pad_to(p['proj2']['W'][60:120], 128, 256),
        _pad_to(p['proj2']['W'][120:180], 128, 256), _bias_pad(p['proj2']['b'], 256),
        _pad_to(p['be1']['W'], 128, 64), _bias_pad(p['be1']['b'], 64),
        _pad_to(p['be2']['W'], 64, 128), _bias_pad(p['be2']['b'], 128),
        wat1[:256], _pad_to(wat1[256:384], 128, 256), _bias_pad(p['at1']['b'], 256),
        _pad_to(p['at2']['W'], 256, 128), _bias_pad(p['at2']['b'], 128),
        p['op1']['W'], _bias_pad(p['op1']['b'], 256),
        p['op2']['W'], _bias_pad(p['op2']['b'], 256),
        p['fi']['W'],
        p['rb1a']['W'], _bias_pad(p['rb1a']['b'], 384),
        p['rb1b']['W'], _bias_pad(p['rb1b']['b'], 384),
        p['rb1r']['W'], _bias_pad(p['rb1r']['b'], 384),
        p['rb2a']['W'], _bias_pad(p['rb2a']['b'], 256),
        p['rb2b']['W'], _bias_pad(p['rb2b']['b'], 256),
        p['rb2r']['W'], _bias_pad(p['rb2r']['b'], 256),
        _pad_to(p['out']['W'], 256, 128), _bias_pad(p['out']['b'], 128),
    ]
    BM = 512
    full = lambda a: pl.BlockSpec(a.shape, lambda i: tuple(0 for _ in a.shape))
    row = lambda c: pl.BlockSpec((BM, c), lambda i: (i, 0))
    in_specs = [row(72)] + [row(128)] * 5 + [full(a) for a in args[6:]]
    seg, feat, xf = pl.pallas_call(
        _h2_kernel,
        grid=(M // BM,),
        in_specs=in_specs,
        out_specs=[row(128), row(256), row(256)],
        out_shape=[jax.ShapeDtypeStruct((M, 128), jnp.float32),
                   jax.ShapeDtypeStruct((M, 256), jnp.float32),
                   jax.ShapeDtypeStruct((M, 256), jnp.float32)],
    )(*args)
    return seg, feat, xf


def _head_kernel(xf_ref, wfi_ref,
                 w1a_ref, b1a_ref, w1b_ref, b1b_ref, w1r_ref, b1r_ref,
                 w2a_ref, b2a_ref, w2b_ref, b2b_ref, w2r_ref, b2r_ref,
                 wo_ref, bo_ref,
                 seg_ref, feat_ref):
    xf = xf_ref[...]
    xg = xf * jax.nn.sigmoid(jnp.dot(xf, wfi_ref[...], preferred_element_type=jnp.float32))
    h = jax.nn.relu(jnp.dot(xg, w1a_ref[...], preferred_element_type=jnp.float32) + b1a_ref[...])
    r1 = jax.nn.relu(jnp.dot(h, w1b_ref[...], preferred_element_type=jnp.float32) + b1b_ref[...])
    r1 = r1 + jnp.dot(xg, w1r_ref[...], preferred_element_type=jnp.float32) + b1r_ref[...]
    h2 = jax.nn.relu(jnp.dot(r1, w2a_ref[...], preferred_element_type=jnp.float32) + b2a_ref[...])
    feat = jax.nn.relu(jnp.dot(h2, w2b_ref[...], preferred_element_type=jnp.float32) + b2b_ref[...])
    feat = feat + jnp.dot(r1, w2r_ref[...], preferred_element_type=jnp.float32) + b2r_ref[...]
    feat_ref[...] = feat
    seg_ref[...] = jnp.dot(feat, wo_ref[...], preferred_element_type=jnp.float32) + bo_ref[...]


def _run_head(x_fused, params):
    B, N = x_fused.shape[0], x_fused.shape[1]
    M = B * N
    xf = x_fused.reshape(M, 256)
    p = params
    wo = jnp.zeros((256, 128), jnp.float32).at[:, :17].set(p['out']['W'])
    bo = jnp.zeros((1, 128), jnp.float32).at[0, :17].set(p['out']['b'])
    BM = 1024
    grid = (M // BM,)
    row_spec = lambda c: pl.BlockSpec((BM, c), lambda i: (i, 0))
    args = [xf,
            p['fi']['W'],
            p['rb1a']['W'], p['rb1a']['b'].reshape(1, -1),
            p['rb1b']['W'], p['rb1b']['b'].reshape(1, -1),
            p['rb1r']['W'], p['rb1r']['b'].reshape(1, -1),
            p['rb2a']['W'], p['rb2a']['b'].reshape(1, -1),
            p['rb2b']['W'], p['rb2b']['b'].reshape(1, -1),
            p['rb2r']['W'], p['rb2r']['b'].reshape(1, -1),
            wo, bo]
    full = lambda a: pl.BlockSpec(a.shape, lambda i: tuple(0 for _ in a.shape))
    in_specs = [row_spec(256)] + [full(a) for a in args[1:]]
    seg, feat = pl.pallas_call(
        _head_kernel,
        grid=grid,
        in_specs=in_specs,
        out_specs=[row_spec(128), row_spec(256)],
        out_shape=[jax.ShapeDtypeStruct((M, 128), jnp.float32),
                   jax.ShapeDtypeStruct((M, 256), jnp.float32)],
    )(*args)
    return seg[:, :17].reshape(B, N, 17), feat.reshape(B, N, 256)


def kernel(x, pos, labels, params):
    B, N = x.shape[0], x.shape[1]
    cd = _cdist(pos, pos)
    sidx = _argsort_rows(cd.reshape(B * N, N))
    knn32 = sidx[:, :32].reshape(B, N, 32)
    dil200 = sidx[:, ::6][:, :32].reshape(B, N, 32)
    dil900 = sidx[:, ::28][:, :32].reshape(B, N, 32)
    dil1800 = sidx[:, ::56][:, :32].reshape(B, N, 32)
    nidx = sidx[:, 1:9].reshape(B, N, 8)
    M = B * N
    x = _stn(params['stn'], x)
    x1p = _edge_conv_fused(params['e1'], x, _flat_idx(knn32))
    x1 = x1p[:, :24].reshape(B, N, 24)
    x2p = _edge_conv_fused(params['e2'], x1, _flat_idx(_top32_idx(_cdist(x1, x1))))
    x2 = x2p[:, :24].reshape(B, N, 24)
    x3p = _edge_conv_fused(params['e3'], x2, _flat_idx(_top32_idx(_cdist(x2, x2))))
    x_local = jnp.concatenate([x1p[:, :24], x2p[:, :24], x3p[:, :24]], -1)
    xm_p = _run_mid(x_local, params)
    x_mid = xm_p[:, :60].reshape(B, N, 60)
    xd1_p = _edge_conv_fused(params['d1'], x_mid, _flat_idx(dil200))
    xd2_p = _edge_conv_fused(params['d2'], xd1_p[:, :60].reshape(B, N, 60), _flat_idx(dil900))
    xd3_p = _edge_conv_fused(params['d3'], xd2_p[:, :60].reshape(B, N, 60), _flat_idx(dil1800))
    h1 = _run_h1(xm_p, xd1_p, xd2_p, xd3_p, params)
    tl = h1[:, 0].astype(jnp.int32).reshape(B, N)
    nl = jax.vmap(lambda lb, ib: lb[ib])(tl, nidx)
    diff = jnp.mean((nl != tl[:, :, None]).astype(jnp.float32), -1).reshape(M, 1)
    binfo = jnp.concatenate([diff, h1[:, 1:3], jnp.zeros((M, 125), jnp.float32)], 1)
    seg, feat, xf = _run_h2(x_local, xm_p, xd1_p, xd2_p, xd3_p, binfo, params)
    return (seg[:, :17].reshape(B, N, 17), feat.reshape(B, N, 256), xf.reshape(B, N, 256))


# speedup vs baseline: 1.0281x; 1.0007x over previous
import functools

import jax
import jax.numpy as jnp
import numpy as np
from jax import lax
from jax.experimental import pallas as pl
from jax.experimental.pallas import tpu as pltpu
from jax.experimental.pallas import tpu_sc as plsc


def _ap(p, x):
    y = x @ p['W']
    if p['b'] is not None:
        y = y + p['b']
    return y


def _cdist(a, b):
    d2 = jnp.sum(a * a, -1)[:, :, None] + jnp.sum(b * b, -1)[:, None, :] - 2.0 * jnp.einsum('bnd,bmd->bnm', a, b)
    return jnp.sqrt(jnp.maximum(d2, 0.0))


def _roll3(x, sh, axis):
    if axis == 1:
        return jnp.concatenate([x[:, sh:, :], x[:, :sh, :]], axis=1)
    return pltpu.roll(x, x.shape[2] - sh, 2)


def _partner3(x, j, ij0):
    L = x.shape[2]
    if j < L:
        lo = _roll3(x, j, 2)
        hi = _roll3(x, L - j, 2)
    else:
        m = j // L
        lo = _roll3(x, m, 1)
        hi = _roll3(x, x.shape[1] - m, 1)
    return jnp.where(ij0, lo, hi)


def _sort_kernel3(n_mid, n_lane, k_ref, v_out):
    k = k_ref[...]
    R = k.shape[0]
    shape = (R, n_mid, n_lane)
    lane = jax.lax.broadcasted_iota(jnp.int32, shape, 2)
    mid = jax.lax.broadcasted_iota(jnp.int32, shape, 1)
    iota = mid * n_lane + lane
    v = iota
    n = n_mid * n_lane
    ksz = 2
    while ksz <= n:
        j = ksz // 2
        while j >= 1:
            ij0 = (iota & j) == 0
            up = (iota & ksz) == 0
            tm = ij0 == up
            kp = _partner3(k, j, ij0)
            vp = _partner3(v, j, ij0)
            le = (k < kp) | ((k == kp) & (v < vp))
            keep = le == tm
            k = jnp.where(keep, k, kp)
            v = jnp.where(keep, v, vp)
            j //= 2
        ksz *= 2
    v_out[...] = v


def _argsort_rows(keys, block_rows=256, n_lane=128):
    M, n = keys.shape
    n_mid = n // n_lane
    k3 = keys.reshape(M, n_mid, n_lane)
    out = pl.pallas_call(
        functools.partial(_sort_kernel3, n_mid, n_lane),
        grid=(M // block_rows,),
        in_specs=[pl.BlockSpec((block_rows, n_mid, n_lane), lambda i: (i, 0, 0))],
        out_specs=pl.BlockSpec((block_rows, n_mid, n_lane), lambda i: (i, 0, 0)),
        out_shape=jax.ShapeDtypeStruct((M, n_mid, n_lane), jnp.int32),
    )(k3)
    return out.reshape(M, n)


def _sc_gather(table, idx):
    V, D = table.shape
    E = idx.shape[0]
    info = plsc.get_sparse_core_info()
    nw = info.num_cores * info.num_subcores
    b_per_w = E // nw
    ch = min(512, b_per_w)
    n_ch = b_per_w // ch
    mesh = plsc.VectorSubcoreMesh(core_axis_name="c", subcore_axis_name="s")

    @functools.partial(
        pl.kernel, mesh=mesh,
        out_type=jax.ShapeDtypeStruct((E, D), jnp.float32),
        scratch_types=[
            pltpu.VMEM((ch,), jnp.int32),
            pltpu.VMEM((ch, D), jnp.float32),
            pltpu.SemaphoreType.DMA,
        ],
    )
    def k(table_hbm, idx_hbm, out_hbm, idx_v, rows_v, sem):
        wid = lax.axis_index("s") * info.num_cores + lax.axis_index("c")
        base = wid * b_per_w
        for c in range(n_ch):
            off = base + c * ch
            pltpu.sync_copy(idx_hbm.at[pl.ds(off, ch)], idx_v)
            pltpu.async_copy(table_hbm.at[idx_v], rows_v, sem).wait()
            pltpu.sync_copy(rows_v, out_hbm.at[pl.ds(off, ch)])

    return k(table, idx)


def _ab_kernel(x_ref, wa_ref, wb_ref, b_ref, a_ref, bm_ref):
    xx = x_ref[...]
    a_ref[...] = jnp.dot(xx, wa_ref[...], preferred_element_type=jnp.float32) + b_ref[...]
    bm_ref[...] = jnp.dot(xx, wb_ref[...], preferred_element_type=jnp.float32)


def _edge_mlp_kernel(a_ref, g_ref, w2_ref, b2_ref, out_ref):
    BM, H = a_ref.shape
    O = w2_ref.shape[1]
    g = g_ref[...]
    e3 = g.reshape(BM, 32, H) + a_ref[...][:, None, :]
    e = jax.nn.relu(e3).reshape(BM * 32, H)
    h = jnp.dot(e, w2_ref[...], preferred_element_type=jnp.float32) + b2_ref[...]
    out_ref[...] = jnp.max(jax.nn.relu(h).reshape(BM, 32, O), axis=1)


def _pad_to(w, r, c):
    return jnp.zeros((r, c), jnp.float32).at[:w.shape[0], :w.shape[1]].set(w)


def _edge_conv_fused(p, x, idx_glob):
    B, N, C = x.shape
    H = p['m1']['W'].shape[1]
    O = p['m2']['W'].shape[1]
    M = B * N
    W1 = p['m1']['W']
    W1a, W1b = W1[:C], W1[C:]
    wa = _pad_to(W1a - W1b, C, 128)
    wb = _pad_to(W1b, C, 128)
    b1p = jnp.zeros((1, 128), jnp.float32).at[0, :H].set(p['m1']['b'])
    a, bm = pl.pallas_call(
        _ab_kernel,
        in_specs=[pl.BlockSpec((M, C), lambda: (0, 0)),
                  pl.BlockSpec((C, 128), lambda: (0, 0)),
                  pl.BlockSpec((C, 128), lambda: (0, 0)),
                  pl.BlockSpec((1, 128), lambda: (0, 0))],
        out_specs=[pl.BlockSpec((M, 128), lambda: (0, 0)),
                   pl.BlockSpec((M, 128), lambda: (0, 0))],
        out_shape=[jax.ShapeDtypeStruct((M, 128), jnp.float32),
                   jax.ShapeDtypeStruct((M, 128), jnp.float32)],
    )(x.reshape(M, C), wa, wb, b1p)
    g = _sc_gather(bm, idx_glob)
    w2p = _pad_to(p['m2']['W'], 128, 128)
    b2p = jnp.zeros((1, 128), jnp.float32).at[0, :O].set(p['m2']['b'])
    BM = 256
    out = pl.pallas_call(
        _edge_mlp_kernel,
        grid=(M // BM,),
        in_specs=[pl.BlockSpec((BM, 128), lambda i: (i, 0)),
                  pl.BlockSpec((BM * 32, 128), lambda i: (i, 0)),
                  pl.BlockSpec((128, 128), lambda i: (0, 0)),
                  pl.BlockSpec((1, 128), lambda i: (0, 0))],
        out_specs=pl.BlockSpec((BM, 128), lambda i: (i, 0)),
        out_shape=jax.ShapeDtypeStruct((M, 128), jnp.float32),
    )(a, g, w2p, b2p)
    return out


def _flat_idx(idx):
    B, N, K = idx.shape
    off = (jnp.arange(B, dtype=jnp.int32) * N)[:, None, None]
    return (idx.astype(jnp.int32) + off).reshape(B * N * K)


def _stn(p, x):
    h = jax.nn.relu(_ap(p['c1'], x))
    h = jax.nn.relu(_ap(p['c2'], h))
    h = jax.nn.relu(_ap(p['c3'], h))
    g = jnp.max(h, axis=1)
    g = jax.nn.relu(_ap(p['f1'], g))
    g = jax.nn.relu(_ap(p['f2'], g))
    t = _ap(p['f3'], g) + jnp.eye(24, dtype=jnp.float32).reshape(-1)
    t = t.reshape(-1, 24, 24)
    return jnp.einsum('bnc,bcd->bnd', x, t)


def _top32_kernel(n_mid, n_lane, k_ref, out_ref):
    k = k_ref[...]
    R = k.shape[0]
    shape = (R, n_mid, n_lane)
    lane = jax.lax.broadcasted_iota(jnp.int32, shape, 2)
    mid = jax.lax.broadcasted_iota(jnp.int32, shape, 1)
    v = mid * n_lane + lane
    big = jnp.int32(1 << 30)
    lane2 = jax.lax.broadcasted_iota(jnp.int32, (R, 128), 1)
    outv = jnp.zeros((R, 128), jnp.int32)
    for t in range(32):
        m = jnp.min(jnp.min(k, axis=2, keepdims=True), axis=1, keepdims=True)
        cand = jnp.where(k == m, v, big)
        vm = jnp.min(jnp.min(cand, axis=2, keepdims=True), axis=1, keepdims=True)
        outv = jnp.where(lane2 == t, jnp.broadcast_to(vm[:, :, 0], (R, 128)), outv)
        k = jnp.where(v == vm, jnp.inf, k)
    out_ref[...] = outv


def _top32_idx(cdm, block_rows=128, n_lane=128):
    B, N, _ = cdm.shape
    M = B * N
    n_mid = N // n_lane
    k3 = cdm.reshape(M, n_mid, n_lane)
    out = pl.pallas_call(
        functools.partial(_top32_kernel, n_mid, n_lane),
        grid=(M // block_rows,),
        in_specs=[pl.BlockSpec((block_rows, n_mid, n_lane), lambda i: (i, 0, 0))],
        out_specs=pl.BlockSpec((block_rows, 128), lambda i: (i, 0)),
        out_shape=jax.ShapeDtypeStruct((M, 128), jnp.int32),
    )(k3)
    return out[:, :32].reshape(B, N, 32)


def _mid_kernel(xl_ref, w_ref, b_ref, out_ref):
    out_ref[...] = jax.nn.relu(
        jnp.dot(xl_ref[...], w_ref[...], preferred_element_type=jnp.float32) + b_ref[...])


def _h1_kernel(xm_ref, x1_ref, x2_ref, x3_ref,
               w0_ref, w1_ref, w2_ref, w3_ref, bt_ref,
               g_ref, lb_ref, wt2_ref, bt2_ref, out_ref):
    dot = lambda a, w: jnp.dot(a, w, preferred_element_type=jnp.float32)
    t = (dot(xm_ref[...], w0_ref[...]) + dot(x1_ref[...], w1_ref[...])
         + dot(x2_ref[...], w2_ref[...]) + dot(x3_ref[...], w3_ref[...]) + bt_ref[...])
    m = jnp.mean(t, -1, keepdims=True)
    v = jnp.mean((t - m) * (t - m), -1, keepdims=True)
    t = (t - m) / jnp.sqrt(v + 1e-5) * g_ref[...] + lb_ref[...]
    l = dot(jax.nn.relu(t), wt2_ref[...]) + bt2_ref[...]
    BM = l.shape[0]
    lane = jax.lax.broadcasted_iota(jnp.int32, (BM, 128), 1)
    l = jnp.where(lane < 17, l, -1e30)
    mx = jnp.max(l, -1, keepdims=True)
    tl = jnp.min(jnp.where(l == mx, lane, 1 << 30), -1, keepdims=True)
    p = jnp.exp(l - mx)
    probs = p / jnp.sum(p, -1, keepdims=True)
    conf = jnp.max(probs, -1, keepdims=True)
    ent = -jnp.sum(probs * jnp.log(probs + 1e-8), -1, keepdims=True) / np.log(17.0)
    out = jnp.where(lane == 0, tl.astype(jnp.float32),
                    jnp.where(lane == 1, conf, jnp.where(lane == 2, ent, 0.0)))
    out_ref[...] = out


def _h2_kernel(xl_ref, xm_ref, x1_ref, x2_ref, x3_ref, bi_ref,
               wp0_ref, bp0_ref, wp1_ref, bp1_ref,
               wp2a_ref, wp2b_ref, wp2c_ref, bp2_ref,
               wbe1_ref, bbe1_ref, wbe2_ref, bbe2_ref,
               wat1a_ref, wat1b_ref, bat1_ref, wat2_ref, bat2_ref,
               wop1_ref, bop1_ref, wop2_ref, bop2_ref,
               wfi_ref,
               w1a_ref, b1a_ref, w1b_ref, b1b_ref, w1r_ref, b1r_ref,
               w2a_ref, b2a_ref, w2b_ref, b2b_ref, w2r_ref, b2r_ref,
               wo_ref, bo_ref,
               seg_ref, feat_ref, xf_ref):
    dot = lambda a, w: jnp.dot(a, w, preferred_element_type=jnp.float32)
    f0 = dot(xl_ref[...], wp0_ref[...]) + bp0_ref[...]
    f1 = dot(xm_ref[...], wp1_ref[...]) + bp1_ref[...]
    f2 = (dot(x1_ref[...], wp2a_ref[...]) + dot(x2_ref[...], wp2b_ref[...])
          + dot(x3_ref[...], wp2c_ref[...]) + bp2_ref[...])
    gfeat = (f0 + f1 + f2) / 3.0
    benc = dot(jax.nn.relu(dot(bi_ref[...], wbe1_ref[...]) + bbe1_ref[...]), wbe2_ref[...]) + bbe2_ref[...]
    a1 = jax.nn.relu(dot(gfeat, wat1a_ref[...]) + dot(benc, wat1b_ref[...]) + bat1_ref[...])
    al = dot(a1, wat2_ref[...]) + bat2_ref[...]
    BM = al.shape[0]
    lane = jax.lax.broadcasted_iota(jnp.int32, (BM, 128), 1)
    al = jnp.where(lane < 3, al, -1e30)
    p = jnp.exp(al - jnp.max(al, -1, keepdims=True))
    aw = p / jnp.sum(p, -1, keepdims=True)
    pick = lambda k: jnp.sum(jnp.where(lane == k, aw, 0.0), -1, keepdims=True)
    fused = f0 * pick(0) + f1 * pick(1) + f2 * pick(2)
    xf = dot(jax.nn.relu(dot(fused, wop1_ref[...]) + bop1_ref[...]), wop2_ref[...]) + bop2_ref[...] + gfeat
    xf_ref[...] = xf
    xg = xf * jax.nn.sigmoid(dot(xf, wfi_ref[...]))
    h = jax.nn.relu(dot(xg, w1a_ref[...]) + b1a_ref[...])
    r1 = jax.nn.relu(dot(h, w1b_ref[...]) + b1b_ref[...])
    r1 = r1 + dot(xg, w1r_ref[...]) + b1r_ref[...]
    h2 = jax.nn.relu(dot(r1, w2a_ref[...]) + b2a_ref[...])
    feat = jax.nn.relu(dot(h2, w2b_ref[...]) + b2b_ref[...])
    feat = feat + dot(r1, w2r_ref[...]) + b2r_ref[...]
    feat_ref[...] = feat
    seg_ref[...] = dot(feat, wo_ref[...]) + bo_ref[...]


def _bias_pad(b, n=128):
    return jnp.zeros((1, n), jnp.float32).at[0, :b.shape[0]].set(b)


def _run_mid(x_local, p):
    M = x_local.shape[0]
    w = _pad_to(p['local_hidden']['W'], 72, 128)
    b = _bias_pad(p['local_hidden']['b'])
    BM = 512
    return pl.pallas_call(
        _mid_kernel,
        grid=(M // BM,),
        in_specs=[pl.BlockSpec((BM, 72), lambda i: (i, 0)),
                  pl.BlockSpec((72, 128), lambda i: (0, 0)),
                  pl.BlockSpec((1, 128), lambda i: (0, 0))],
        out_specs=pl.BlockSpec((BM, 128), lambda i: (i, 0)),
        out_shape=jax.ShapeDtypeStruct((M, 128), jnp.float32),
    )(x_local, w, b)


def _run_h1(xm_p, xd1_p, xd2_p, xd3_p, p):
    M = xm_p.shape[0]
    wt = p['temp1']['W']
    w0 = _pad_to(wt[:60], 128, 128)
    w1 = _pad_to(wt[60:120], 128, 128)
    w2 = _pad_to(wt[120:180], 128, 128)
    w3 = _pad_to(wt[180:240], 128, 128)
    bt = _bias_pad(p['temp1']['b'])
    g = _bias_pad(p['temp_ln']['g'])
    lb = _bias_pad(p['temp_ln']['b'])
    wt2 = _pad_to(p['temp2']['W'], 128, 128)
    bt2 = _bias_pad(p['temp2']['b'])
    BM = 512
    full = lambda a: pl.BlockSpec(a.shape, lambda i: tuple(0 for _ in a.shape))
    args = [xm_p, xd1_p, xd2_p, xd3_p, w0, w1, w2, w3, bt, g, lb, wt2, bt2]
    in_specs = [pl.BlockSpec((BM, 128), lambda i: (i, 0))] * 4 + [full(a) for a in args[4:]]
    return pl.pallas_call(
        _h1_kernel,
        grid=(M // BM,),
        in_specs=in_specs,
        out_specs=pl.BlockSpec((BM, 128), lambda i: (i, 0)),
        out_shape=jax.ShapeDtypeStruct((M, 128), jnp.float32),
    )(*args)


def _run_h2(x_local, xm_p, xd1_p, xd2_p, xd3_p, binfo, p):
    M = x_local.shape[0]
    wat1 = p['at1']['W']
    args = [
        x_local, xm_p, xd1_p, xd2_p, xd3_p, binfo,
        p['proj0']['W'], _bias_pad(p['proj0']['b'], 256),
        _pad_to(p['proj1']['W'], 128, 256), _bias_pad(p['proj1']['b'], 256),
        _pad_to(p['proj2']['W'][:60], 128, 256), _pad_to(p['proj2']['W'][60:120], 128, 256),
        _pad_to(p['proj2']['W'][120:180], 128, 256), _bias_pad(p['proj2']['b'], 256),
        _pad_to(p['be1']['W'], 128, 64), _bias_pad(p['be1']['b'], 64),
        _pad_to(p['be2']['W'], 64, 128), _bias_pad(p['be2']['b'], 128),
        wat1[:256], _pad_to(wat1[256:384], 128, 256), _bias_pad(p['at1']['b'], 256),
        _pad_to(p['at2']['W'], 256, 128), _bias_pad(p['at2']['b'], 128),
        p['op1']['W'], _bias_pad(p['op1']['b'], 256),
        p['op2']['W'], _bias_pad(p['op2']['b'], 256),
        p['fi']['W'],
        p['rb1a']['W'], _bias_pad(p['rb1a']['b'], 384),
        p['rb1b']['W'], _bias_pad(p['rb1b']['b'], 384),
        p['rb1r']['W'], _bias_pad(p['rb1r']['b'], 384),
        p['rb2a']['W'], _bias_pad(p['rb2a']['b'], 256),
        p['rb2b']['W'], _bias_pad(p['rb2b']['b'], 256),
        p['rb2r']['W'], _bias_pad(p['rb2r']['b'], 256),
        _pad_to(p['out']['W'], 256, 128), _bias_pad(p['out']['b'], 128),
    ]
    BM = 512
    full = lambda a: pl.BlockSpec(a.shape, lambda i: tuple(0 for _ in a.shape))
    row = lambda c: pl.BlockSpec((BM, c), lambda i: (i, 0))
    in_specs = [row(72)] + [row(128)] * 5 + [full(a) for a in args[6:]]
    seg, feat, xf = pl.pallas_call(
        _h2_kernel,
        grid=(M // BM,),
        in_specs=in_specs,
        out_specs=[row(128), row(256), row(256)],
        out_shape=[jax.ShapeDtypeStruct((M, 128), jnp.float32),
                   jax.ShapeDtypeStruct((M, 256), jnp.float32),
                   jax.ShapeDtypeStruct((M, 256), jnp.float32)],
    )(*args)
    return seg, feat, xf


def kernel(x, pos, labels, params):
    B, N = x.shape[0], x.shape[1]
    cd = _cdist(pos, pos)
    sidx = _argsort_rows(cd.reshape(B * N, N))
    knn32 = sidx[:, :32].reshape(B, N, 32)
    dil200 = sidx[:, ::6][:, :32].reshape(B, N, 32)
    dil900 = sidx[:, ::28][:, :32].reshape(B, N, 32)
    dil1800 = sidx[:, ::56][:, :32].reshape(B, N, 32)
    nidx = sidx[:, 1:9].reshape(B, N, 8)
    M = B * N
    x = _stn(params['stn'], x)
    x1p = _edge_conv_fused(params['e1'], x, _flat_idx(knn32))
    x1 = x1p[:, :24].reshape(B, N, 24)
    x2p = _edge_conv_fused(params['e2'], x1, _flat_idx(_top32_idx(_cdist(x1, x1))))
    x2 = x2p[:, :24].reshape(B, N, 24)
    x3p = _edge_conv_fused(params['e3'], x2, _flat_idx(_top32_idx(_cdist(x2, x2))))
    x_local = jnp.concatenate([x1p[:, :24], x2p[:, :24], x3p[:, :24]], -1)
    xm_p = _run_mid(x_local, params)
    x_mid = xm_p[:, :60].reshape(B, N, 60)
    xd1_p = _edge_conv_fused(params['d1'], x_mid, _flat_idx(dil200))
    xd2_p = _edge_conv_fused(params['d2'], xd1_p[:, :60].reshape(B, N, 60), _flat_idx(dil900))
    xd3_p = _edge_conv_fused(params['d3'], xd2_p[:, :60].reshape(B, N, 60), _flat_idx(dil1800))
    h1 = _run_h1(xm_p, xd1_p, xd2_p, xd3_p, params)
    tl = h1[:, 0].astype(jnp.int32).reshape(B, N)
    nl = jax.vmap(lambda lb, ib: lb[ib])(tl, nidx)
    diff = jnp.mean((nl != tl[:, :, None]).astype(jnp.float32), -1).reshape(M, 1)
    binfo = jnp.concatenate([diff, h1[:, 1:3], jnp.zeros((M, 125), jnp.float32)], 1)
    seg, feat, xf = _run_h2(x_local, xm_p, xd1_p, xd2_p, xd3_p, binfo, params)
    return (seg[:, :17].reshape(B, N, 17), feat.reshape(B, N, 256), xf.reshape(B, N, 256))
